# baseline (speedup 1.0000x reference)
import functools

import numpy as np
import jax
import jax.numpy as jnp
from jax import lax
from jax.experimental import pallas as pl
from jax.experimental.pallas import tpu as pltpu
from jax.experimental.pallas import tpu_sc as plsc

D_MODEL = 64
MAX_POS = 128
SCALE = 8.0

NUM_CORES = 2
NUM_SUBCORES = 16
NUM_WORKERS = NUM_CORES * NUM_SUBCORES
CHUNK = 128
NBUF = 2


def _pos_encoding_np():
    position = np.arange(MAX_POS)[:, np.newaxis]
    k = np.arange(D_MODEL)[np.newaxis, :]
    i = k // 2
    angle_rates = 1 / np.power(10000, 2 * i / np.float32(D_MODEL))
    angle_rads = position * angle_rates
    angle_rads[:, 0::2] = np.sin(angle_rads[:, 0::2])
    angle_rads[:, 1::2] = np.cos(angle_rads[:, 1::2])
    return angle_rads.astype(np.float32)


_POS_T = np.ascontiguousarray(_pos_encoding_np().T)


@functools.partial(jax.jit, static_argnames=("n_seq",))
def _sc_embed(x2d, pos_t, table, *, n_seq):
    seq_per_w = n_seq // NUM_WORKERS
    n_rounds = seq_per_w // NBUF

    mesh = plsc.VectorSubcoreMesh(core_axis_name="c", subcore_axis_name="s")

    @functools.partial(
        pl.kernel,
        mesh=mesh,
        compiler_params=pltpu.CompilerParams(
            use_tc_tiling_on_sc=True, needs_layout_passes=False),
        out_type=jax.ShapeDtypeStruct((n_seq, D_MODEL, MAX_POS), jnp.float32),
        scratch_types=(
            [pltpu.VMEM((seq_per_w, CHUNK), jnp.int32)]
            + [pltpu.VMEM((D_MODEL, MAX_POS), jnp.float32)]
            + [pltpu.VMEM((CHUNK, D_MODEL), jnp.float32)] * NBUF
            + [pltpu.VMEM((D_MODEL, MAX_POS), jnp.float32)] * NBUF
            + [pltpu.SemaphoreType.DMA] * (2 * NBUF)
        ),
    )
    def k(x_hbm, pos_hbm, table_hbm, out_hbm, idx_v, pos_v, *bufs):
        rows = bufs[:NBUF]
        outs = bufs[NBUF:2 * NBUF]
        gsem = bufs[2 * NBUF:3 * NBUF]
        osem = bufs[3 * NBUF:4 * NBUF]

        wid = lax.axis_index("s") * NUM_CORES + lax.axis_index("c")
        w_seq = wid * seq_per_w
        pltpu.sync_copy(pos_hbm, pos_v)
        pltpu.sync_copy(x_hbm.at[pl.ds(w_seq, seq_per_w), :], idx_v)

        def fire_rows(c, b):
            def grp(g, carry):
                v = idx_v[c, pl.ds(g * 16, 16)]
                for t in range(16):
                    pltpu.async_copy(
                        table_hbm.at[v[t]], rows[b].at[g * 16 + t], gsem[b])
                return carry
            lax.fori_loop(0, CHUNK // 16, grp, 0, unroll=False)

        def wait_rows(b):
            for kk in range(CHUNK // 16):
                pltpu.make_async_copy(
                    table_hbm.at[pl.ds(0, 16)],
                    rows[b].at[pl.ds(kk * 16, 16)], gsem[b]).wait()

        for b in range(NBUF):
            fire_rows(b, b)

        def round_body(r, carry):
            for b in range(NBUF):
                c = r * NBUF + b
                wait_rows(b)

                @pl.when(r > 0)
                def _():
                    pltpu.make_async_copy(
                        outs[b], out_hbm.at[w_seq + c], osem[b]).wait()

                def col_body(f, carry2):
                    for g in range(MAX_POS // 16):
                        sl = pl.ds(g * 16, 16)
                        ids = lax.iota(jnp.int32, 16) + g * 16
                        col = jnp.full((16,), f, jnp.int32)
                        v = plsc.load_gather(rows[b], [ids, col])
                        outs[b][f, sl] = v * SCALE + pos_v[f, sl]
                    return carry2

                lax.fori_loop(0, D_MODEL, col_body, 0, unroll=2)

                pltpu.async_copy(outs[b], out_hbm.at[w_seq + c], osem[b])

                @pl.when(r < n_rounds - 1)
                def _():
                    fire_rows(c + NBUF, b)
            return carry

        lax.fori_loop(0, n_rounds, round_body, 0, unroll=False)

        for b in range(NBUF):
            c = (n_rounds - 1) * NBUF + b
            pltpu.make_async_copy(
                outs[b], out_hbm.at[w_seq + c], osem[b]).wait()

    return k(x2d, pos_t, table)


def kernel(x, table):
    b, s = x.shape
    pos_t = jnp.asarray(_POS_T)
    out = _sc_embed(x, pos_t, table, n_seq=b)
    return out.transpose(0, 2, 1)
